# baseline (speedup 1.0000x reference)
import jax
import jax.numpy as jnp
from jax import lax
from jax.experimental import pallas as pl
from jax.experimental.pallas import tpu as pltpu
from jax.experimental.pallas import tpu_sc as plsc

B = 16384
D = 64
DP = 128
NE = 1_000_000
NR = 1000
NC = 2
NS = 16
NW = NC * NS
BPW = B // NW
NG_FULL = NE // DP
TAIL = NE - NG_FULL * DP
GPW = (NG_FULL + NW - 1) // NW
NRG_FULL = NR // DP
RTAIL = NR - NRG_FULL * DP


def _transpose_block(src, dst, ncols):
    dvec = lax.iota(jnp.int32, 16)

    @pl.loop(0, ncols)
    def _cols(c):
        cv = jnp.full((16,), c, jnp.int32)
        for k in range(D // 16):
            x = plsc.load_gather(src, [dvec + k * 16, cv])
            dst[c, pl.ds(k * 16, 16)] = x


def _format_body(ent_t, rel_t, ent_tail, rel_tail, ent_out, rel_out,
                 gbuf, tbuf, rsem):
    wid = lax.axis_index("s") * NC + lax.axis_index("c")

    @pl.loop(0, GPW)
    def _groups(k):
        g = wid * GPW + k

        @pl.when(g < NG_FULL)
        def _():
            pltpu.sync_copy(ent_t.at[:, pl.ds(g * DP, DP)], gbuf)
            _transpose_block(gbuf, tbuf, DP)
            pltpu.sync_copy(tbuf, ent_out.at[pl.ds(g * DP, DP), :])

    @pl.when(wid == NW - 1)
    def _ent_tail():
        pltpu.sync_copy(ent_tail.at[:],
                        ent_out.at[pl.ds(NG_FULL * DP, TAIL), :])

    for rg in range(NRG_FULL):
        @pl.when(wid == rg)
        def _rel_group(rg=rg):
            pltpu.sync_copy(rel_t.at[:, pl.ds(rg * DP, DP)], gbuf)
            _transpose_block(gbuf, tbuf, DP)
            pltpu.sync_copy(tbuf, rel_out.at[pl.ds(rg * DP, DP), :])

    @pl.when(wid == NRG_FULL)
    def _rel_tail():
        pltpu.sync_copy(rel_tail.at[:],
                        rel_out.at[pl.ds(NRG_FULL * DP, RTAIL), :])


_mesh = plsc.VectorSubcoreMesh(core_axis_name="c", subcore_axis_name="s")

_format = pl.kernel(
    _format_body,
    mesh=_mesh,
    out_type=(jax.ShapeDtypeStruct((NE, DP), jnp.float32),
              jax.ShapeDtypeStruct((NR, DP), jnp.float32)),
    scratch_types=[
        pltpu.VMEM((D, DP), jnp.float32),
        pltpu.VMEM((DP, DP), jnp.float32),
        pltpu.SemaphoreType.DMA,
    ],
    compiler_params=pltpu.CompilerParams(use_tc_tiling_on_sc=True,
                                         needs_layout_passes=False),
)


def _gather6_body(h_i, r_i, t_i, he_i, re_i, te_i, ent, rel,
                  o0, o1, o2, o3, o4, o5,
                  idx_v, rows_v, sem):
    wid = lax.axis_index("s") * NC + lax.axis_index("c")
    base = wid * BPW
    tasks = ((h_i, ent, o0), (r_i, rel, o1), (t_i, ent, o2),
             (he_i, ent, o3), (re_i, rel, o4), (te_i, ent, o5))
    for idx_hbm, table, out_hbm in tasks:
        pltpu.sync_copy(idx_hbm.at[pl.ds(base, BPW)], idx_v)
        pltpu.async_copy(table.at[idx_v], rows_v, sem).wait()
        pltpu.sync_copy(rows_v, out_hbm.at[pl.ds(base, BPW)])


_gather6 = pl.kernel(
    _gather6_body,
    mesh=_mesh,
    out_type=tuple(jax.ShapeDtypeStruct((B, DP), jnp.float32) for _ in range(6)),
    scratch_types=[
        pltpu.VMEM((BPW,), jnp.int32),
        pltpu.VMEM((BPW, DP), jnp.float32),
        pltpu.SemaphoreType.DMA,
    ],
    compiler_params=pltpu.CompilerParams(use_tc_tiling_on_sc=True),
)


def kernel(pos_head, pos_rel, pos_tail, pos_head_exp, pos_rel_exp,
           pos_tail_exp, entity_table, rel_table):
    idxs = [jnp.asarray(x, jnp.int32) for x in
            (pos_head, pos_rel, pos_tail, pos_head_exp, pos_rel_exp, pos_tail_exp)]
    ent_tail = jnp.pad(entity_table[NG_FULL * DP:], ((0, 0), (0, DP - D)))
    rel_tail = jnp.pad(rel_table[NRG_FULL * DP:], ((0, 0), (0, DP - D)))
    ent_fmt, rel_fmt = _format(entity_table.T, rel_table.T, ent_tail, rel_tail)
    outs = _gather6(*idxs, ent_fmt, rel_fmt)
    return tuple(o[:, :D] for o in outs)
